# baseline (speedup 1.0000x reference)
import functools

import jax
import jax.numpy as jnp
from jax import lax
from jax.experimental import pallas as pl
from jax.experimental.pallas import tpu as pltpu
from jax.experimental.pallas import tpu_sc as plsc

BATCH = 16384
EMB = 32
NUM_WORKERS = 32
BPW = BATCH // NUM_WORKERS


def _sc_gather_body(uid, iid, gu, gi, mu, mi,
                    o_gu, o_gi, o_mu, o_mi,
                    idx_u, idx_i, r0, r1, r2, r3, s0, s1, s2, s3):
    c = lax.axis_index("c")
    s = lax.axis_index("s")
    wid = s * 2 + c
    base = wid * BPW
    pltpu.sync_copy(uid.at[pl.ds(base, BPW)], idx_u)
    pltpu.sync_copy(iid.at[pl.ds(base, BPW)], idx_i)
    cp0 = pltpu.async_copy(gu.at[idx_u], r0, s0)
    cp1 = pltpu.async_copy(gi.at[idx_i], r1, s1)
    cp2 = pltpu.async_copy(mu.at[idx_u], r2, s2)
    cp3 = pltpu.async_copy(mi.at[idx_i], r3, s3)
    cp0.wait()
    pltpu.sync_copy(r0, o_gu.at[pl.ds(base, BPW)])
    cp1.wait()
    pltpu.sync_copy(r1, o_gi.at[pl.ds(base, BPW)])
    cp2.wait()
    pltpu.sync_copy(r2, o_mu.at[pl.ds(base, BPW)])
    cp3.wait()
    pltpu.sync_copy(r3, o_mi.at[pl.ds(base, BPW)])


def _sc_gather(user_id, item_id, gu, gi, mu, mi):
    mesh = plsc.VectorSubcoreMesh(core_axis_name="c", subcore_axis_name="s")
    row = jax.ShapeDtypeStruct((BATCH, EMB), jnp.float32)
    fn = pl.kernel(
        _sc_gather_body,
        mesh=mesh,
        compiler_params=pltpu.CompilerParams(use_tc_tiling_on_sc=False),
        out_type=[row, row, row, row],
        scratch_types=[
            pltpu.VMEM((BPW,), jnp.int32),
            pltpu.VMEM((BPW,), jnp.int32),
            pltpu.VMEM((BPW, EMB), jnp.float32),
            pltpu.VMEM((BPW, EMB), jnp.float32),
            pltpu.VMEM((BPW, EMB), jnp.float32),
            pltpu.VMEM((BPW, EMB), jnp.float32),
            pltpu.SemaphoreType.DMA,
            pltpu.SemaphoreType.DMA,
            pltpu.SemaphoreType.DMA,
            pltpu.SemaphoreType.DMA,
        ],
    )
    return fn(user_id, item_id, gu, gi, mu, mi)


BB = 2048


def _mlp_body(gu_ref, gi_ref, mu_ref, mi_ref, w1u_ref, w1i_ref, b1_ref,
              w2_ref, b2_ref, wg_ref, wm_ref, bout_ref, out_ref):
    h = jnp.maximum(
        jnp.dot(mu_ref[...], w1u_ref[...], preferred_element_type=jnp.float32)
        + jnp.dot(mi_ref[...], w1i_ref[...], preferred_element_type=jnp.float32)
        + b1_ref[...], 0.0)
    m = jnp.maximum(
        jnp.dot(h, w2_ref[...], preferred_element_type=jnp.float32)
        + b2_ref[...], 0.0)
    g = gu_ref[...] * gi_ref[...]
    out_ref[...] = (
        jnp.dot(g, wg_ref[...], preferred_element_type=jnp.float32)
        + jnp.dot(m, wm_ref[...], preferred_element_type=jnp.float32)
        + bout_ref[0, 0])


def _tc_mlp(gu_e, gi_e, mu_e, mi_e, W1, b1, W2, b2, Wout, bout):
    w1u = W1[:EMB, :]
    w1i = W1[EMB:, :]
    wg = Wout[:EMB, :]
    wm = Wout[EMB:, :]
    b1r = b1.reshape(1, -1)
    b2r = b2.reshape(1, -1)
    boutr = bout.reshape(1, 1)
    grid = (BATCH // BB,)
    blk = lambda i: (i, 0)
    full = lambda i: (0, 0)
    return pl.pallas_call(
        _mlp_body,
        grid=grid,
        in_specs=[
            pl.BlockSpec((BB, EMB), blk),
            pl.BlockSpec((BB, EMB), blk),
            pl.BlockSpec((BB, EMB), blk),
            pl.BlockSpec((BB, EMB), blk),
            pl.BlockSpec((EMB, 32), full),
            pl.BlockSpec((EMB, 32), full),
            pl.BlockSpec((1, 32), full),
            pl.BlockSpec((32, 16), full),
            pl.BlockSpec((1, 16), full),
            pl.BlockSpec((EMB, 1), full),
            pl.BlockSpec((16, 1), full),
            pl.BlockSpec((1, 1), full),
        ],
        out_specs=pl.BlockSpec((BB, 1), blk),
        out_shape=jax.ShapeDtypeStruct((BATCH, 1), jnp.float32),
    )(gu_e, gi_e, mu_e, mi_e, w1u, w1i, b1r, W2, b2r, wg, wm, boutr)


def kernel(user_id, item_id, gmf_user_table, gmf_item_table,
           mlp_user_table, mlp_item_table, W1, b1, W2, b2, Wout, bout):
    gu_e, gi_e, mu_e, mi_e = _sc_gather(
        user_id.astype(jnp.int32), item_id.astype(jnp.int32),
        gmf_user_table, gmf_item_table, mlp_user_table, mlp_item_table)
    return _tc_mlp(gu_e, gi_e, mu_e, mi_e, W1, b1, W2, b2, Wout, bout)


# speedup vs baseline: 3.6592x; 3.6592x over previous
import functools

import jax
import jax.numpy as jnp
from jax import lax
from jax.experimental import pallas as pl
from jax.experimental.pallas import tpu as pltpu
from jax.experimental.pallas import tpu_sc as plsc

BATCH = 16384
EMB = 32
NROWS = 1000000
NW = 32
BPW = BATCH // NW
W = 128
SLOTS = 4


def _sc_gather_body(cu, lu, ci, li, gut, git, mut, mit,
                    o_g, o_mu, o_mi,
                    vcu, vlu, vci, vli,
                    sgu, sgi, smu, smi,
                    gst, must, mist, *sems):
    c = lax.axis_index("c")
    s = lax.axis_index("s")
    wid = s * 2 + c
    base = wid * BPW
    pltpu.sync_copy(cu.at[pl.ds(base, BPW)], vcu)
    pltpu.sync_copy(lu.at[pl.ds(base, BPW)], vlu)
    pltpu.sync_copy(ci.at[pl.ds(base, BPW)], vci)
    pltpu.sync_copy(li.at[pl.ds(base, BPW)], vli)

    iota = lax.iota(jnp.int32, 16)
    g0 = iota >> 3
    e0 = iota & 7

    def issue(cun, cin, k):
        cun = pl.multiple_of(cun, W)
        cin = pl.multiple_of(cin, W)
        pltpu.make_async_copy(
            gut.at[:, :, pl.ds(cun, W)], sgu.at[k], sems[k]).start()
        pltpu.make_async_copy(
            git.at[:, :, pl.ds(cin, W)], sgi.at[k], sems[k]).start()
        pltpu.make_async_copy(
            mut.at[:, :, pl.ds(cun, W)], smu.at[k], sems[k]).start()
        pltpu.make_async_copy(
            mit.at[:, :, pl.ds(cin, W)], smi.at[k], sems[k]).start()

    def drain(k):
        for buf in (sgu, sgi, smu, smi):
            pltpu.make_async_copy(
                gut.at[:, :, pl.ds(0, W)], buf.at[k], sems[k]).wait()

    def extract(lun, lin, n, k):
        lu16 = jnp.full((16,), lun, jnp.int32)
        li16 = jnp.full((16,), lin, jnp.int32)
        for h in range(2):
            gv = g0 + 2 * h
            u = plsc.load_gather(sgu.at[k], [gv, e0, lu16])
            i = plsc.load_gather(sgi.at[k], [gv, e0, li16])
            gst[pl.ds(n * EMB + 16 * h, 16)] = u * i
            mu_v = plsc.load_gather(smu.at[k], [gv, e0, lu16])
            mi_v = plsc.load_gather(smi.at[k], [gv, e0, li16])
            must[pl.ds(n * EMB + 16 * h, 16)] = mu_v
            mist[pl.ds(n * EMB + 16 * h, 16)] = mi_v

    ngroups = BPW // 16
    cu0 = vcu[pl.ds(0, 16)]
    ci0 = vci[pl.ds(0, 16)]
    lu0 = vlu[pl.ds(0, 16)]
    li0 = vli[pl.ds(0, 16)]
    for k in range(SLOTS):
        issue(cu0[k], ci0[k], k)

    def body(g, carry):
        cu_c, ci_c, lu_c, li_c = carry
        nxt = jnp.minimum(g + 1, ngroups - 1) * 16
        cu_n = vcu[pl.ds(nxt, 16)]
        ci_n = vci[pl.ds(nxt, 16)]
        lu_n = vlu[pl.ds(nxt, 16)]
        li_n = vli[pl.ds(nxt, 16)]
        for k in range(16):
            n = g * 16 + k
            slot = k % SLOTS
            drain(slot)
            extract(lu_c[k], li_c[k], n, slot)
            nn = n + SLOTS
            kk = k + SLOTS
            cun = cu_c[kk] if kk < 16 else cu_n[kk - 16]
            cin = ci_c[kk] if kk < 16 else ci_n[kk - 16]

            @pl.when(nn < BPW)
            def _():
                issue(cun, cin, slot)
        return (cu_n, ci_n, lu_n, li_n)

    lax.fori_loop(0, ngroups, body, (cu0, ci0, lu0, li0))

    pltpu.sync_copy(gst, o_g.at[pl.ds(base * EMB, BPW * EMB)])
    pltpu.sync_copy(must, o_mu.at[pl.ds(base * EMB, BPW * EMB)])
    pltpu.sync_copy(mist, o_mi.at[pl.ds(base * EMB, BPW * EMB)])


def _sc_gather(cu, lu, ci, li, gut, git, mut, mit):
    mesh = plsc.VectorSubcoreMesh(core_axis_name="c", subcore_axis_name="s")
    out = jax.ShapeDtypeStruct((BATCH * EMB,), jnp.float32)
    slab = pltpu.VMEM((SLOTS, 4, 8, W), jnp.float32)
    fn = pl.kernel(
        _sc_gather_body,
        mesh=mesh,
        compiler_params=pltpu.CompilerParams(
            use_tc_tiling_on_sc=True, needs_layout_passes=False),
        out_type=[out, out, out],
        scratch_types=[
            pltpu.VMEM((BPW,), jnp.int32),
            pltpu.VMEM((BPW,), jnp.int32),
            pltpu.VMEM((BPW,), jnp.int32),
            pltpu.VMEM((BPW,), jnp.int32),
            slab, slab, slab, slab,
            pltpu.VMEM((BPW * EMB,), jnp.float32),
            pltpu.VMEM((BPW * EMB,), jnp.float32),
            pltpu.VMEM((BPW * EMB,), jnp.float32),
        ] + [pltpu.SemaphoreType.DMA] * SLOTS,
    )
    return fn(cu, lu, ci, li, gut, git, mut, mit)


BB = 2048


def _mlp_body(g_ref, mu_ref, mi_ref, w1u_ref, w1i_ref, b1_ref,
              w2_ref, b2_ref, wg_ref, wm_ref, bout_ref, out_ref):
    h = jnp.maximum(
        jnp.dot(mu_ref[...], w1u_ref[...], preferred_element_type=jnp.float32)
        + jnp.dot(mi_ref[...], w1i_ref[...], preferred_element_type=jnp.float32)
        + b1_ref[...], 0.0)
    m = jnp.maximum(
        jnp.dot(h, w2_ref[...], preferred_element_type=jnp.float32)
        + b2_ref[...], 0.0)
    out_ref[...] = (
        jnp.dot(g_ref[...], wg_ref[...], preferred_element_type=jnp.float32)
        + jnp.dot(m, wm_ref[...], preferred_element_type=jnp.float32)
        + bout_ref[0, 0])


def _tc_mlp(g_e, mu_e, mi_e, W1, b1, W2, b2, Wout, bout):
    w1u = W1[:EMB, :]
    w1i = W1[EMB:, :]
    wg = Wout[:EMB, :]
    wm = Wout[EMB:, :]
    b1r = b1.reshape(1, -1)
    b2r = b2.reshape(1, -1)
    boutr = bout.reshape(1, 1)
    grid = (BATCH // BB,)
    blk = lambda i: (i, 0)
    full = lambda i: (0, 0)
    return pl.pallas_call(
        _mlp_body,
        grid=grid,
        in_specs=[
            pl.BlockSpec((BB, EMB), blk),
            pl.BlockSpec((BB, EMB), blk),
            pl.BlockSpec((BB, EMB), blk),
            pl.BlockSpec((EMB, 32), full),
            pl.BlockSpec((EMB, 32), full),
            pl.BlockSpec((1, 32), full),
            pl.BlockSpec((32, 16), full),
            pl.BlockSpec((1, 16), full),
            pl.BlockSpec((EMB, 1), full),
            pl.BlockSpec((16, 1), full),
            pl.BlockSpec((1, 1), full),
        ],
        out_specs=pl.BlockSpec((BB, 1), blk),
        out_shape=jax.ShapeDtypeStruct((BATCH, 1), jnp.float32),
    )(g_e, mu_e, mi_e, w1u, w1i, b1r, W2, b2r, wg, wm, boutr)


def kernel(user_id, item_id, gmf_user_table, gmf_item_table,
           mlp_user_table, mlp_item_table, W1, b1, W2, b2, Wout, bout):
    uid = user_id.astype(jnp.int32)
    iid = item_id.astype(jnp.int32)
    cu = (uid // W) * W
    lu = uid - cu
    ci = (iid // W) * W
    li = iid - ci
    gut = gmf_user_table.T.reshape(4, 8, NROWS)
    git = gmf_item_table.T.reshape(4, 8, NROWS)
    mut = mlp_user_table.T.reshape(4, 8, NROWS)
    mit = mlp_item_table.T.reshape(4, 8, NROWS)
    g_f, mu_f, mi_f = _sc_gather(cu, lu, ci, li, gut, git, mut, mit)
    g_e = g_f.reshape(BATCH, EMB)
    mu_e = mu_f.reshape(BATCH, EMB)
    mi_e = mi_f.reshape(BATCH, EMB)
    return _tc_mlp(g_e, mu_e, mi_e, W1, b1, W2, b2, Wout, bout)
